# baseline (speedup 1.0000x reference)
import functools

import jax
import jax.numpy as jnp
from jax import lax
from jax.experimental import pallas as pl
from jax.experimental.pallas import tpu as pltpu
from jax.experimental.pallas import tpu_sc as plsc

N = 10000
D = 128
E = 320000
NC = 2
NS = 16
NW = NC * NS
CH = 128
G = 16
NG = 5
NCHUNK = G * NG
E_PAD = NW * CH * NCHUNK
N_PAD = 10112
RPT = N_PAD // NS

_mesh = plsc.VectorSubcoreMesh(core_axis_name="c", subcore_axis_name="s")


@functools.partial(
    pl.kernel,
    mesh=_mesh,
    out_type=jax.ShapeDtypeStruct((NC, N_PAD, 16), jnp.float32),
    scratch_types=[
        pltpu.VMEM((NCHUNK, CH), jnp.int32),
        pltpu.VMEM((CH, 16), jnp.float32),
        pltpu.SemaphoreType.DMA,
        pltpu.VMEM_SHARED((N_PAD, 16), jnp.float32),
    ],
)
def _deg_kernel(dst_h, zero_h, ones_h, out_h, dst_v, ones_v, sem, acc):
    c = lax.axis_index("c")
    s = lax.axis_index("s")
    wid = s * NC + c
    base = s * RPT
    pltpu.sync_copy(zero_h.at[pl.ds(base, RPT)], acc.at[pl.ds(base, RPT)])
    pltpu.sync_copy(ones_h, ones_v)
    pltpu.sync_copy(dst_h.at[wid], dst_v)
    plsc.subcore_barrier()

    def chunk(j, carry):
        pltpu.sync_copy(ones_v, acc.at[dst_v.at[j]], add=True)
        return carry

    lax.fori_loop(0, NCHUNK, chunk, 0)
    plsc.subcore_barrier()
    pltpu.sync_copy(acc.at[pl.ds(base, RPT)], out_h.at[c, pl.ds(base, RPT)])


@functools.partial(
    pl.kernel,
    mesh=_mesh,
    out_type=jax.ShapeDtypeStruct((NC, N_PAD, D), jnp.float32),
    scratch_types=[
        pltpu.VMEM((NCHUNK, CH), jnp.int32),
        pltpu.VMEM((NCHUNK, CH), jnp.int32),
        pltpu.VMEM((CH, D), jnp.float32),
        pltpu.SemaphoreType.DMA,
        pltpu.VMEM_SHARED((N_PAD, D), jnp.float32),
    ],
)
def _agg_kernel(y_h, src_h, dst_h, zero_h, out_h, sidx, didx, rows0, sem0, acc):
    c = lax.axis_index("c")
    s = lax.axis_index("s")
    wid = s * NC + c
    base = s * RPT

    @pl.when(c == 0)
    def _():
        pltpu.sync_copy(y_h.at[pl.ds(base, RPT)], acc.at[pl.ds(base, RPT)])

    @pl.when(c == 1)
    def _():
        pltpu.sync_copy(zero_h.at[pl.ds(base, RPT)], acc.at[pl.ds(base, RPT)])

    pltpu.sync_copy(src_h.at[wid], sidx)
    pltpu.sync_copy(dst_h.at[wid], didx)
    plsc.subcore_barrier()

    def chunk(j, carry):
        pltpu.async_copy(y_h.at[sidx.at[j]], rows0, sem0).wait()
        pltpu.sync_copy(rows0, acc.at[didx.at[j]], add=True)
        return carry

    lax.fori_loop(0, NCHUNK, chunk, 0)
    plsc.subcore_barrier()
    pltpu.sync_copy(acc.at[pl.ds(base, RPT)], out_h.at[c, pl.ds(base, RPT)])


_R = 1000


def _row_spec(w):
    return pl.BlockSpec((_R, w), lambda i: (i, 0))


def _rep_spec(h, w):
    return pl.BlockSpec((h, w), lambda i: (0, 0))


def _dis(d0_ref, d1_ref):
    return lax.rsqrt(1.0 + d0_ref[:, 0:1] + d1_ref[:, 0:1])


def _y1_body(x_ref, w_ref, d0_ref, d1_ref, o_ref):
    xw = jnp.dot(x_ref[...], w_ref[...], preferred_element_type=jnp.float32)
    o_ref[...] = xw * _dis(d0_ref, d1_ref)


def _y2_body(p0_ref, p1_ref, d0_ref, d1_ref, w_ref, b_ref, o_ref):
    dis = _dis(d0_ref, d1_ref)
    h = jnp.maximum(dis * (p0_ref[...] + p1_ref[...]) + b_ref[...], 0.0)
    o_ref[...] = jnp.dot(h, w_ref[...], preferred_element_type=jnp.float32) * dis


def _out_body(p0_ref, p1_ref, d0_ref, d1_ref, b_ref, o_ref):
    dis = _dis(d0_ref, d1_ref)
    o_ref[...] = jnp.maximum(dis * (p0_ref[...] + p1_ref[...]) + b_ref[...], 0.0)


def _tc_y1(x, W1, d0, d1):
    return pl.pallas_call(
        _y1_body,
        grid=(N // _R,),
        in_specs=[_row_spec(D), _rep_spec(D, D), _row_spec(16), _row_spec(16)],
        out_specs=_row_spec(D),
        out_shape=jax.ShapeDtypeStruct((N, D), jnp.float32),
    )(x, W1, d0, d1)


def _tc_y2(p0, p1, d0, d1, W2, b1):
    return pl.pallas_call(
        _y2_body,
        grid=(N // _R,),
        in_specs=[_row_spec(D), _row_spec(D), _row_spec(16), _row_spec(16),
                  _rep_spec(D, D), _rep_spec(1, D)],
        out_specs=_row_spec(D),
        out_shape=jax.ShapeDtypeStruct((N, D), jnp.float32),
    )(p0, p1, d0, d1, W2, b1)


def _tc_out(p0, p1, d0, d1, b2):
    return pl.pallas_call(
        _out_body,
        grid=(N // _R,),
        in_specs=[_row_spec(D), _row_spec(D), _row_spec(16), _row_spec(16),
                  _rep_spec(1, D)],
        out_specs=_row_spec(D),
        out_shape=jax.ShapeDtypeStruct((N, D), jnp.float32),
    )(p0, p1, d0, d1, b2)


def kernel(x, edge_index, W1, b1, W2, b2):
    src = edge_index[0].astype(jnp.int32)
    dst = edge_index[1].astype(jnp.int32)
    pad = E_PAD - E
    src_t = jnp.concatenate([src, jnp.zeros((pad,), jnp.int32)]).reshape(NW, NCHUNK, CH)
    dst_t = jnp.concatenate([dst, jnp.full((pad,), N, jnp.int32)]).reshape(NW, NCHUNK, CH)

    z16 = jnp.zeros((N_PAD, 16), jnp.float32)
    ones16 = jnp.ones((CH, 16), jnp.float32)
    zbig = jnp.zeros((N_PAD, D), jnp.float32)

    degp = _deg_kernel(dst_t, z16, ones16)
    d0 = degp[0, :N]
    d1 = degp[1, :N]

    y1 = _tc_y1(x, W1, d0, d1)
    y1p = jnp.pad(y1, ((0, N_PAD - N), (0, 0)))
    agg1 = _agg_kernel(y1p, src_t, dst_t, zbig)

    y2 = _tc_y2(agg1[0, :N], agg1[1, :N], d0, d1, W2, b1.reshape(1, D))
    y2p = jnp.pad(y2, ((0, N_PAD - N), (0, 0)))
    agg2 = _agg_kernel(y2p, src_t, dst_t, zbig)

    return _tc_out(agg2[0, :N], agg2[1, :N], d0, d1, b2.reshape(1, D))


# speedup vs baseline: 1.0014x; 1.0014x over previous
import functools

import jax
import jax.numpy as jnp
from jax import lax
from jax.experimental import pallas as pl
from jax.experimental.pallas import tpu as pltpu
from jax.experimental.pallas import tpu_sc as plsc

N = 10000
D = 128
E = 320000
NC = 2
NS = 16
NW = NC * NS
CH = 128
G = 16
NG = 5
NCHUNK = G * NG
E_PAD = NW * CH * NCHUNK
N_PAD = 10112
RPT = N_PAD // NS

_mesh = plsc.VectorSubcoreMesh(core_axis_name="c", subcore_axis_name="s")


@functools.partial(
    pl.kernel,
    mesh=_mesh,
    out_type=jax.ShapeDtypeStruct((NC, N_PAD, 16), jnp.float32),
    scratch_types=[
        pltpu.VMEM((NCHUNK, CH), jnp.int32),
        pltpu.VMEM((CH, 16), jnp.float32),
        pltpu.SemaphoreType.DMA,
        pltpu.VMEM_SHARED((N_PAD, 16), jnp.float32),
    ],
)
def _deg_kernel(dst_h, zero_h, ones_h, out_h, dst_v, ones_v, sem, acc):
    c = lax.axis_index("c")
    s = lax.axis_index("s")
    wid = s * NC + c
    base = s * RPT
    pltpu.sync_copy(zero_h.at[pl.ds(base, RPT)], acc.at[pl.ds(base, RPT)])
    pltpu.sync_copy(ones_h, ones_v)
    pltpu.sync_copy(dst_h.at[wid], dst_v)
    plsc.subcore_barrier()

    def chunk(j, carry):
        pltpu.sync_copy(ones_v, acc.at[dst_v.at[j]], add=True)
        return carry

    lax.fori_loop(0, NCHUNK, chunk, 0)
    plsc.subcore_barrier()
    pltpu.sync_copy(acc.at[pl.ds(base, RPT)], out_h.at[c, pl.ds(base, RPT)])


@functools.partial(
    pl.kernel,
    mesh=_mesh,
    out_type=jax.ShapeDtypeStruct((NC, N_PAD, D), jnp.float32),
    scratch_types=[
        pltpu.VMEM((NCHUNK, CH), jnp.int32),
        pltpu.VMEM((NCHUNK, CH), jnp.int32),
        pltpu.VMEM((CH, D), jnp.float32),
        pltpu.SemaphoreType.DMA,
        pltpu.VMEM_SHARED((N_PAD, D), jnp.float32),
    ],
)
def _agg_kernel(y_h, src_h, dst_h, zero_h, out_h, sidx, didx, rows0, sem0, acc):
    c = lax.axis_index("c")
    s = lax.axis_index("s")
    wid = s * NC + c
    base = s * RPT

    @pl.when(c == 0)
    def _():
        pltpu.sync_copy(y_h.at[pl.ds(base, RPT)], acc.at[pl.ds(base, RPT)])

    @pl.when(c == 1)
    def _():
        pltpu.sync_copy(zero_h.at[pl.ds(base, RPT)], acc.at[pl.ds(base, RPT)])

    pltpu.sync_copy(src_h.at[wid], sidx)
    pltpu.sync_copy(dst_h.at[wid], didx)
    plsc.subcore_barrier()

    def chunk(j, carry):
        pltpu.async_copy(y_h.at[sidx.at[j]], rows0, sem0).wait()
        pltpu.sync_copy(rows0, acc.at[didx.at[j]], add=True)
        return carry

    lax.fori_loop(0, NCHUNK, chunk, 0)
    plsc.subcore_barrier()
    pltpu.sync_copy(acc.at[pl.ds(base, RPT)], out_h.at[c, pl.ds(base, RPT)])


_R = 1000


def _row_spec(w):
    return pl.BlockSpec((_R, w), lambda i: (i, 0))


def _rep_spec(h, w):
    return pl.BlockSpec((h, w), lambda i: (0, 0))


def _dis(d0_ref, d1_ref):
    return lax.rsqrt(1.0 + d0_ref[:, 0:1] + d1_ref[:, 0:1])


def _y1_body(x_ref, w_ref, d0_ref, d1_ref, o_ref):
    xw = jnp.dot(x_ref[...], w_ref[...], preferred_element_type=jnp.float32)
    o_ref[...] = xw * _dis(d0_ref, d1_ref)


def _y2_body(p0_ref, p1_ref, d0_ref, d1_ref, w_ref, b_ref, o_ref):
    dis = _dis(d0_ref, d1_ref)
    h = jnp.maximum(dis * (p0_ref[...] + p1_ref[...]) + b_ref[...], 0.0)
    o_ref[...] = jnp.dot(h, w_ref[...], preferred_element_type=jnp.float32) * dis


def _out_body(p0_ref, p1_ref, d0_ref, d1_ref, b_ref, o_ref):
    dis = _dis(d0_ref, d1_ref)
    o_ref[...] = jnp.maximum(dis * (p0_ref[...] + p1_ref[...]) + b_ref[...], 0.0)


def _tc_y1(x, W1, d0, d1):
    return pl.pallas_call(
        _y1_body,
        grid=(N // _R,),
        in_specs=[_row_spec(D), _rep_spec(D, D), _row_spec(16), _row_spec(16)],
        out_specs=_row_spec(D),
        out_shape=jax.ShapeDtypeStruct((N, D), jnp.float32),
    )(x, W1, d0, d1)


def _tc_y2(p0, p1, d0, d1, W2, b1):
    return pl.pallas_call(
        _y2_body,
        grid=(N // _R,),
        in_specs=[_row_spec(D), _row_spec(D), _row_spec(16), _row_spec(16),
                  _rep_spec(D, D), _rep_spec(1, D)],
        out_specs=_row_spec(D),
        out_shape=jax.ShapeDtypeStruct((N, D), jnp.float32),
    )(p0, p1, d0, d1, W2, b1)


def _tc_out(p0, p1, d0, d1, b2):
    return pl.pallas_call(
        _out_body,
        grid=(N // _R,),
        in_specs=[_row_spec(D), _row_spec(D), _row_spec(16), _row_spec(16),
                  _rep_spec(1, D)],
        out_specs=_row_spec(D),
        out_shape=jax.ShapeDtypeStruct((N, D), jnp.float32),
    )(p0, p1, d0, d1, b2)


def kernel(x, edge_index, W1, b1, W2, b2):
    src = edge_index[0].astype(jnp.int32)
    dst = edge_index[1].astype(jnp.int32)
    pad = E_PAD - E
    src_t = jnp.concatenate([src, jnp.zeros((pad,), jnp.int32)]).reshape(NW, NCHUNK, CH)
    pad_dst = N + jnp.arange(pad, dtype=jnp.int32) % (N_PAD - N)
    dst_t = jnp.concatenate([dst, pad_dst]).reshape(NW, NCHUNK, CH)

    z16 = jnp.zeros((N_PAD, 16), jnp.float32)
    ones16 = jnp.ones((CH, 16), jnp.float32)
    zbig = jnp.zeros((N_PAD, D), jnp.float32)

    degp = _deg_kernel(dst_t, z16, ones16)
    d0 = degp[0, :N]
    d1 = degp[1, :N]

    y1 = _tc_y1(x, W1, d0, d1)
    y1p = jnp.pad(y1, ((0, N_PAD - N), (0, 0)))
    agg1 = _agg_kernel(y1p, src_t, dst_t, zbig)

    y2 = _tc_y2(agg1[0, :N], agg1[1, :N], d0, d1, W2, b1.reshape(1, D))
    y2p = jnp.pad(y2, ((0, N_PAD - N), (0, 0)))
    agg2 = _agg_kernel(y2p, src_t, dst_t, zbig)

    return _tc_out(agg2[0, :N], agg2[1, :N], d0, d1, b2.reshape(1, D))


# speedup vs baseline: 1.4961x; 1.4940x over previous
import functools

import jax
import jax.numpy as jnp
from jax import lax
from jax.experimental import pallas as pl
from jax.experimental.pallas import tpu as pltpu
from jax.experimental.pallas import tpu_sc as plsc

N = 10000
D = 128
E = 320000
NC = 2
NS = 16
NW = NC * NS
CH = 128
NCHUNK = -(-E // (NW * CH))
E_PAD = NW * CH * NCHUNK
N_PAD = 10112
RPT = N_PAD // NS

_mesh = plsc.VectorSubcoreMesh(core_axis_name="c", subcore_axis_name="s")


@functools.partial(
    pl.kernel,
    mesh=_mesh,
    out_type=jax.ShapeDtypeStruct((NC, N_PAD, 16), jnp.float32),
    scratch_types=[
        pltpu.VMEM((NCHUNK, CH), jnp.int32),
        pltpu.VMEM((CH, 16), jnp.float32),
        pltpu.SemaphoreType.DMA,
        pltpu.VMEM_SHARED((N_PAD, 16), jnp.float32),
    ],
)
def _deg_kernel(dst_h, zero_h, ones_h, out_h, dst_v, ones_v, sem, acc):
    c = lax.axis_index("c")
    s = lax.axis_index("s")
    wid = s * NC + c
    base = s * RPT
    pltpu.sync_copy(zero_h.at[pl.ds(base, RPT)], acc.at[pl.ds(base, RPT)])
    pltpu.sync_copy(ones_h, ones_v)
    pltpu.sync_copy(dst_h.at[wid], dst_v)
    plsc.subcore_barrier()

    def chunk(j, carry):
        pltpu.sync_copy(ones_v, acc.at[dst_v.at[j]], add=True)
        return carry

    lax.fori_loop(0, NCHUNK, chunk, 0)
    plsc.subcore_barrier()
    pltpu.sync_copy(acc.at[pl.ds(base, RPT)], out_h.at[c, pl.ds(base, RPT)])


@functools.partial(
    pl.kernel,
    mesh=_mesh,
    out_type=jax.ShapeDtypeStruct((NC, N_PAD, D), jnp.float32),
    scratch_types=[
        pltpu.VMEM((NCHUNK, CH), jnp.int32),
        pltpu.VMEM((NCHUNK, CH), jnp.int32),
        pltpu.VMEM((CH, D), jnp.float32),
        pltpu.SemaphoreType.DMA,
        pltpu.VMEM_SHARED((N_PAD, D), jnp.float32),
    ],
)
def _agg_kernel(y_h, src_h, dst_h, zero_h, out_h, sidx, didx, rows0, sem0, acc):
    c = lax.axis_index("c")
    s = lax.axis_index("s")
    wid = s * NC + c
    base = s * RPT

    @pl.when(c == 0)
    def _():
        pltpu.sync_copy(y_h.at[pl.ds(base, RPT)], acc.at[pl.ds(base, RPT)])

    @pl.when(c == 1)
    def _():
        pltpu.sync_copy(zero_h.at[pl.ds(base, RPT)], acc.at[pl.ds(base, RPT)])

    pltpu.sync_copy(src_h.at[wid], sidx)
    pltpu.sync_copy(dst_h.at[wid], didx)
    plsc.subcore_barrier()

    def chunk(j, carry):
        pltpu.async_copy(y_h.at[sidx.at[j]], rows0, sem0).wait()
        pltpu.sync_copy(rows0, acc.at[didx.at[j]], add=True)
        return carry

    lax.fori_loop(0, NCHUNK, chunk, 0)
    plsc.subcore_barrier()
    pltpu.sync_copy(acc.at[pl.ds(base, RPT)], out_h.at[c, pl.ds(base, RPT)])


_R = 1000


def _row_spec(w):
    return pl.BlockSpec((_R, w), lambda i: (i, 0))


def _rep_spec(h, w):
    return pl.BlockSpec((h, w), lambda i: (0, 0))


def _dis(d0_ref, d1_ref):
    return lax.rsqrt(1.0 + d0_ref[:, 0:1] + d1_ref[:, 0:1])


def _y1_body(x_ref, w_ref, d0_ref, d1_ref, o_ref):
    xw = jnp.dot(x_ref[...], w_ref[...], preferred_element_type=jnp.float32)
    o_ref[...] = xw * _dis(d0_ref, d1_ref)


def _y2_body(p0_ref, p1_ref, d0_ref, d1_ref, w_ref, b_ref, o_ref):
    dis = _dis(d0_ref, d1_ref)
    h = jnp.maximum(dis * (p0_ref[...] + p1_ref[...]) + b_ref[...], 0.0)
    o_ref[...] = jnp.dot(h, w_ref[...], preferred_element_type=jnp.float32) * dis


def _out_body(p0_ref, p1_ref, d0_ref, d1_ref, b_ref, o_ref):
    dis = _dis(d0_ref, d1_ref)
    o_ref[...] = jnp.maximum(dis * (p0_ref[...] + p1_ref[...]) + b_ref[...], 0.0)


def _tc_y1(x, W1, d0, d1):
    return pl.pallas_call(
        _y1_body,
        grid=(N // _R,),
        in_specs=[_row_spec(D), _rep_spec(D, D), _row_spec(16), _row_spec(16)],
        out_specs=_row_spec(D),
        out_shape=jax.ShapeDtypeStruct((N, D), jnp.float32),
    )(x, W1, d0, d1)


def _tc_y2(p0, p1, d0, d1, W2, b1):
    return pl.pallas_call(
        _y2_body,
        grid=(N // _R,),
        in_specs=[_row_spec(D), _row_spec(D), _row_spec(16), _row_spec(16),
                  _rep_spec(D, D), _rep_spec(1, D)],
        out_specs=_row_spec(D),
        out_shape=jax.ShapeDtypeStruct((N, D), jnp.float32),
    )(p0, p1, d0, d1, W2, b1)


def _tc_out(p0, p1, d0, d1, b2):
    return pl.pallas_call(
        _out_body,
        grid=(N // _R,),
        in_specs=[_row_spec(D), _row_spec(D), _row_spec(16), _row_spec(16),
                  _rep_spec(1, D)],
        out_specs=_row_spec(D),
        out_shape=jax.ShapeDtypeStruct((N, D), jnp.float32),
    )(p0, p1, d0, d1, b2)


def kernel(x, edge_index, W1, b1, W2, b2):
    src = edge_index[0].astype(jnp.int32)
    dst = edge_index[1].astype(jnp.int32)
    pad = E_PAD - E
    src_t = jnp.concatenate([src, jnp.zeros((pad,), jnp.int32)]).reshape(NW, NCHUNK, CH)
    dst_t = jnp.concatenate([dst, jnp.full((pad,), N, jnp.int32)]).reshape(NW, NCHUNK, CH)

    z16 = jnp.zeros((N_PAD, 16), jnp.float32)
    ones16 = jnp.ones((CH, 16), jnp.float32)
    zbig = jnp.zeros((N_PAD, D), jnp.float32)

    degp = _deg_kernel(dst_t, z16, ones16)
    d0 = degp[0, :N]
    d1 = degp[1, :N]

    y1 = _tc_y1(x, W1, d0, d1)
    y1p = jnp.pad(y1, ((0, N_PAD - N), (0, 0)))
    agg1 = _agg_kernel(y1p, src_t, dst_t, zbig)

    y2 = _tc_y2(agg1[0, :N], agg1[1, :N], d0, d1, W2, b1.reshape(1, D))
    y2p = jnp.pad(y2, ((0, N_PAD - N), (0, 0)))
    agg2 = _agg_kernel(y2p, src_t, dst_t, zbig)

    return _tc_out(agg2[0, :N], agg2[1, :N], d0, d1, b2.reshape(1, D))
